# Initial kernel scaffold; baseline (speedup 1.0000x reference)
import jax
import jax.numpy as jnp
from jax.experimental import pallas as pl


def kernel(pinyin_ids, table):
    raise NotImplementedError("write your pallas kernel here")


# speedup vs baseline: 3.0556x; 3.0556x over previous
import functools

import jax
import jax.numpy as jnp
from jax import lax
from jax.experimental import pallas as pl
from jax.experimental.pallas import tpu as pltpu
from jax.experimental.pallas import tpu_sc as plsc

_D = 128
_CH = 128


@functools.lru_cache(maxsize=None)
def _build(B, V, D):
    info = plsc.get_sparse_core_info()
    nw = info.num_cores * info.num_subcores
    b_per_w = B // nw
    n_ch = b_per_w // _CH
    assert b_per_w * nw == B and n_ch * _CH == b_per_w

    mesh = plsc.VectorSubcoreMesh(core_axis_name="c", subcore_axis_name="s")

    @functools.partial(
        pl.kernel,
        mesh=mesh,
        out_type=jax.ShapeDtypeStruct((B, D), jnp.float32),
        scratch_types=[
            pltpu.VMEM((b_per_w,), jnp.int32),
            pltpu.VMEM((_CH, D), jnp.float32),
            pltpu.SemaphoreType.DMA,
        ],
    )
    def gather_kernel(idx_hbm, table_hbm, out_hbm, idx_v, buf, sem):
        wid = lax.axis_index("s") * info.num_cores + lax.axis_index("c")
        base = wid * b_per_w
        pltpu.sync_copy(idx_hbm.at[pl.ds(base, b_per_w)], idx_v)
        for j in range(n_ch):
            pltpu.async_copy(
                table_hbm.at[idx_v.at[pl.ds(j * _CH, _CH)]], buf, sem
            ).wait()
            pltpu.sync_copy(buf, out_hbm.at[pl.ds(base + j * _CH, _CH)])

    return gather_kernel


def kernel(pinyin_ids, table):
    s0, s1 = pinyin_ids.shape
    V, D = table.shape
    flat = pinyin_ids.reshape(s0 * s1)
    out = _build(s0 * s1, V, D)(flat, table)
    return out.reshape(s0, s1, D)


# speedup vs baseline: 3.3371x; 1.0921x over previous
import functools

import jax
import jax.numpy as jnp
from jax import lax
from jax.experimental import pallas as pl
from jax.experimental.pallas import tpu as pltpu
from jax.experimental.pallas import tpu_sc as plsc

_D = 128
_CH = 128


@functools.lru_cache(maxsize=None)
def _build(B, V, D):
    info = plsc.get_sparse_core_info()
    nw = info.num_cores * info.num_subcores
    b_per_w = B // nw
    n_ch = b_per_w // _CH
    assert b_per_w * nw == B and n_ch * _CH == b_per_w

    mesh = plsc.VectorSubcoreMesh(core_axis_name="c", subcore_axis_name="s")

    nbuf = 6
    depth = 4

    @functools.partial(
        pl.kernel,
        mesh=mesh,
        out_type=jax.ShapeDtypeStruct((B, D), jnp.float32),
        scratch_types=[
            pltpu.VMEM((b_per_w,), jnp.int32),
        ]
        + [pltpu.VMEM((_CH, D), jnp.float32) for _ in range(nbuf)]
        + [pltpu.SemaphoreType.DMA for _ in range(2 * nbuf)],
    )
    def gather_kernel(idx_hbm, table_hbm, out_hbm, idx_v, *scratch):
        bufs = scratch[:nbuf]
        gsems = scratch[nbuf : 2 * nbuf]
        ssems = scratch[2 * nbuf :]
        wid = lax.axis_index("s") * info.num_cores + lax.axis_index("c")
        base = wid * b_per_w
        pltpu.sync_copy(idx_hbm.at[pl.ds(base, b_per_w)], idx_v)

        def gather(j):
            return pltpu.async_copy(
                table_hbm.at[idx_v.at[pl.ds(j * _CH, _CH)]],
                bufs[j % nbuf],
                gsems[j % nbuf],
            )

        gathers = [None] * n_ch
        scatters = [None] * n_ch
        for j in range(min(depth, n_ch)):
            gathers[j] = gather(j)
        for j in range(n_ch):
            gathers[j].wait()
            scatters[j] = pltpu.async_copy(
                bufs[j % nbuf],
                out_hbm.at[pl.ds(base + j * _CH, _CH)],
                ssems[j % nbuf],
            )
            k = j + depth
            if k < n_ch:
                if k >= nbuf:
                    scatters[k - nbuf].wait()
                    scatters[k - nbuf] = None
                gathers[k] = gather(k)
        for j in range(n_ch):
            if scatters[j] is not None:
                scatters[j].wait()

    return gather_kernel


def kernel(pinyin_ids, table):
    s0, s1 = pinyin_ids.shape
    V, D = table.shape
    flat = pinyin_ids.reshape(s0 * s1)
    out = _build(s0 * s1, V, D)(flat, table)
    return out.reshape(s0, s1, D)


# speedup vs baseline: 3.3608x; 1.0071x over previous
import functools

import jax
import jax.numpy as jnp
from jax import lax
from jax.experimental import pallas as pl
from jax.experimental.pallas import tpu as pltpu
from jax.experimental.pallas import tpu_sc as plsc

_D = 128
_CH = 256


@functools.lru_cache(maxsize=None)
def _build(B, V, D):
    info = plsc.get_sparse_core_info()
    nw = info.num_cores * info.num_subcores
    b_per_w = B // nw
    n_ch = b_per_w // _CH
    assert b_per_w * nw == B and n_ch * _CH == b_per_w

    mesh = plsc.VectorSubcoreMesh(core_axis_name="c", subcore_axis_name="s")

    nbuf = 3
    depth = 2

    @functools.partial(
        pl.kernel,
        mesh=mesh,
        out_type=jax.ShapeDtypeStruct((B, D), jnp.float32),
        scratch_types=[
            pltpu.VMEM((b_per_w,), jnp.int32),
        ]
        + [pltpu.VMEM((_CH, D), jnp.float32) for _ in range(nbuf)]
        + [pltpu.SemaphoreType.DMA for _ in range(2 * nbuf)],
    )
    def gather_kernel(idx_hbm, table_hbm, out_hbm, idx_v, *scratch):
        bufs = scratch[:nbuf]
        gsems = scratch[nbuf : 2 * nbuf]
        ssems = scratch[2 * nbuf :]
        wid = lax.axis_index("s") * info.num_cores + lax.axis_index("c")
        base = wid * b_per_w
        pltpu.sync_copy(idx_hbm.at[pl.ds(base, b_per_w)], idx_v)

        def gather(j):
            return pltpu.async_copy(
                table_hbm.at[idx_v.at[pl.ds(j * _CH, _CH)]],
                bufs[j % nbuf],
                gsems[j % nbuf],
            )

        gathers = [None] * n_ch
        scatters = [None] * n_ch
        for j in range(min(depth, n_ch)):
            gathers[j] = gather(j)
        for j in range(n_ch):
            gathers[j].wait()
            scatters[j] = pltpu.async_copy(
                bufs[j % nbuf],
                out_hbm.at[pl.ds(base + j * _CH, _CH)],
                ssems[j % nbuf],
            )
            k = j + depth
            if k < n_ch:
                if k >= nbuf:
                    scatters[k - nbuf].wait()
                    scatters[k - nbuf] = None
                gathers[k] = gather(k)
        for j in range(n_ch):
            if scatters[j] is not None:
                scatters[j].wait()

    return gather_kernel


def kernel(pinyin_ids, table):
    s0, s1 = pinyin_ids.shape
    V, D = table.shape
    flat = pinyin_ids.reshape(s0 * s1)
    out = _build(s0 * s1, V, D)(flat, table)
    return out.reshape(s0, s1, D)
